# baseline (speedup 1.0000x reference)
import functools

import jax
import jax.numpy as jnp
from jax import lax
from jax.experimental import pallas as pl
from jax.experimental.pallas import tpu as pltpu
from jax.experimental.pallas import tpu_sc as plsc

D = 128
K = 10
K1 = K + 1
NW = 32
G = 11
GI = 128
VPR = D // 16
NBUF = 4
GPM = 8


def _sc_aggregate(features, idx_grp, b_pad):
    q_per_w = b_pad // NW
    ng = q_per_w // G
    nm = ng // GPM
    mesh = plsc.VectorSubcoreMesh(core_axis_name="c", subcore_axis_name="s")

    @functools.partial(
        pl.kernel,
        mesh=mesh,
        out_type=jax.ShapeDtypeStruct((b_pad, 2 * D), jnp.float32),
        scratch_types=[
            pltpu.VMEM((ng * GI,), jnp.int32),
            *[pltpu.VMEM((GI, D), jnp.float32) for _ in range(NBUF)],
            *[pltpu.VMEM((GPM * G, 2 * D), jnp.float32) for _ in range(2)],
            pltpu.SemaphoreType.DMA,
            *[pltpu.SemaphoreType.DMA for _ in range(NBUF)],
            *[pltpu.SemaphoreType.DMA for _ in range(2)],
        ],
    )
    def agg(features_hbm, idx_hbm, out_comb, idx_v, *bufs):
        rows = list(bufs[0:NBUF])
        comb = list(bufs[NBUF:NBUF + 2])
        sem_i = bufs[NBUF + 2]
        sem_g = list(bufs[NBUF + 3:NBUF + 3 + NBUF])
        sem_w = list(bufs[NBUF + 3 + NBUF:NBUF + 5 + NBUF])

        wid = lax.axis_index("s") * 2 + lax.axis_index("c")
        qbase = wid * q_per_w
        pltpu.async_copy(idx_hbm.at[pl.ds(wid * (ng * GI), ng * GI)],
                         idx_v, sem_i).wait()

        def gather(n, p):
            pltpu.async_copy(
                features_hbm.at[idx_v.at[pl.ds(n * GI, GI)]],
                rows[p], sem_g[p])

        for p in range(NBUF):
            gather(p, p)

        def pair_iter(i, carry):
            for mp in range(2):
                m = i * 2 + mp
                @pl.when(i > 0)
                def _():
                    pltpu.make_async_copy(
                        comb[mp], out_comb.at[pl.ds(qbase, GPM * G)],
                        sem_w[mp]).wait()
                for g in range(GPM):
                    n = m * GPM + g
                    p = g % NBUF
                    pltpu.make_async_copy(
                        features_hbm.at[idx_v.at[pl.ds(0, GI)]],
                        rows[p], sem_g[p]).wait()

                    def query(q, c2):
                        for v in range(VPR):
                            sl = pl.ds(v * 16, 16)
                            acc = rows[p][q * K1, sl]
                            for k in range(1, K):
                                acc = acc + rows[p][q * K1 + k, sl]
                            comb[mp][g * G + q, sl] = acc
                            comb[mp][g * G + q, pl.ds(D + v * 16, 16)] = \
                                rows[p][q * K1 + K, sl]
                        return c2

                    lax.fori_loop(0, G, query, 0)

                    @pl.when(n + NBUF < ng)
                    def _():
                        gather(n + NBUF, p)
                pltpu.async_copy(comb[mp],
                                 out_comb.at[pl.ds(qbase + m * (GPM * G),
                                                   GPM * G)],
                                 sem_w[mp])
            return carry

        lax.fori_loop(0, nm // 2, pair_iter, 0)
        for mp in range(2):
            pltpu.make_async_copy(comb[mp],
                                  out_comb.at[pl.ds(qbase, GPM * G)],
                                  sem_w[mp]).wait()

    return agg(features, idx_grp)


def _tc_compress(comb, w, b_out):
    bt = 512
    grid = (pl.cdiv(b_out, bt),)

    def body(w_ref, comb_ref, o_ref):
        w_all = w_ref[...]
        col = lax.broadcasted_iota(jnp.int32, (D, 2 * D), 1)
        w_scl = jnp.where(col < D, w_all * jnp.float32(1.0 / K), w_all)
        dn = (((1,), (1,)), ((), ()))
        o_ref[...] = jnp.maximum(
            lax.dot_general(w_scl, comb_ref[...], dn,
                            preferred_element_type=jnp.float32), 0.0)

    return pl.pallas_call(
        body,
        grid=grid,
        in_specs=[
            pl.BlockSpec((D, 2 * D), lambda j: (0, 0)),
            pl.BlockSpec((bt, 2 * D), lambda j: (j, 0)),
        ],
        out_specs=pl.BlockSpec((D, bt), lambda j: (0, j)),
        out_shape=jax.ShapeDtypeStruct((D, b_out), jnp.float32),
    )(w, comb)


def kernel(nodes, neigh_idx, features, W_compress):
    b = nodes.shape[0]
    step = NW * G * GPM * 2
    b_pad = ((b + step - 1) // step) * step
    while ((b + 511) // 512) * 512 > b_pad:
        b_pad += step
    idx_all = jnp.concatenate([neigh_idx, nodes[:, None]], axis=1)
    idx_all = jnp.pad(idx_all, ((0, b_pad - b), (0, 0)))
    idx_grp = jnp.pad(idx_all.reshape(b_pad // G, G * K1),
                      ((0, 0), (0, GI - G * K1)))
    comb = _sc_aggregate(features, idx_grp.reshape(-1), b_pad)
    return _tc_compress(comb, W_compress, b)


# speedup vs baseline: 3.7507x; 3.7507x over previous
import functools

import jax
import jax.numpy as jnp
from jax import lax
from jax.experimental import pallas as pl
from jax.experimental.pallas import tpu as pltpu
from jax.experimental.pallas import tpu_sc as plsc

D = 128
K = 10
K1 = K + 1
NW = 32
G = 8
VPR = D // 16
NBUF = 4
GB = G * K1 * D * 4
CB = G * 2 * D * 4


def _sc_aggregate(features, idx_flat, b_pad):
    q_per_w = b_pad // NW
    ng = q_per_w // G
    nq = ng // NBUF
    mesh = plsc.VectorSubcoreMesh(core_axis_name="c", subcore_axis_name="s")

    @functools.partial(
        pl.kernel,
        mesh=mesh,
        out_type=jax.ShapeDtypeStruct((b_pad, 2 * D), jnp.float32),
        scratch_types=[
            pltpu.VMEM((q_per_w * K1,), jnp.int32),
            *[pltpu.VMEM((G * K1, D), jnp.float32) for _ in range(NBUF)],
            *[pltpu.VMEM((G, 2 * D), jnp.float32) for _ in range(NBUF)],
            pltpu.SemaphoreType.DMA,
            *[pltpu.SemaphoreType.DMA for _ in range(NBUF)],
            *[pltpu.SemaphoreType.DMA for _ in range(NBUF)],
        ],
    )
    def agg(features_hbm, idx_hbm, out_comb, idx_v, *bufs):
        rows = list(bufs[0:NBUF])
        comb = list(bufs[NBUF:2 * NBUF])
        sem_i = bufs[2 * NBUF]
        sem_g = list(bufs[2 * NBUF + 1:2 * NBUF + 1 + NBUF])
        sem_w = list(bufs[2 * NBUF + 1 + NBUF:2 * NBUF + 1 + 2 * NBUF])

        wid = lax.axis_index("s") * 2 + lax.axis_index("c")
        qbase = wid * q_per_w
        pltpu.async_copy(idx_hbm.at[pl.ds(qbase * K1, q_per_w * K1)],
                         idx_v, sem_i).wait()

        def gather(n, p):
            pltpu.async_copy(
                features_hbm.at[idx_v.at[pl.ds(n * (G * K1), G * K1)]],
                rows[p], sem_g[p])

        for p in range(NBUF):
            gather(p, p)

        def ring_iter(i, carry):
            for p in range(NBUF):
                n = i * NBUF + p
                pltpu.make_async_copy(
                    features_hbm.at[pl.ds(0, G * K1)],
                    rows[p], sem_g[p]).wait()
                @pl.when(i > 0)
                def _():
                    pltpu.make_async_copy(
                        comb[p], out_comb.at[pl.ds(qbase, G)],
                        sem_w[p]).wait()
                for q in range(G):
                    for v in range(VPR):
                        sl = pl.ds(v * 16, 16)
                        acc = rows[p][q * K1, sl]
                        for k in range(1, K):
                            acc = acc + rows[p][q * K1 + k, sl]
                        comb[p][q, sl] = acc
                        comb[p][q, pl.ds(D + v * 16, 16)] = \
                            rows[p][q * K1 + K, sl]
                @pl.when(i < nq - 1)
                def _():
                    gather(n + NBUF, p)
                pltpu.async_copy(comb[p],
                                 out_comb.at[pl.ds(qbase + n * G, G)],
                                 sem_w[p])
            return carry

        lax.fori_loop(0, nq, ring_iter, 0)
        for p in range(NBUF):
            pltpu.make_async_copy(comb[p], out_comb.at[pl.ds(qbase, G)],
                                  sem_w[p]).wait()

    return agg(features, idx_flat)


def _tc_compress(comb, w, b_out):
    bt = 512
    grid = (pl.cdiv(b_out, bt),)

    def body(w_ref, comb_ref, o_ref):
        w_all = w_ref[...]
        col = lax.broadcasted_iota(jnp.int32, (D, 2 * D), 1)
        w_scl = jnp.where(col < D, w_all * jnp.float32(1.0 / K), w_all)
        dn = (((1,), (1,)), ((), ()))
        o_ref[...] = jnp.maximum(
            lax.dot_general(w_scl, comb_ref[...], dn,
                            preferred_element_type=jnp.float32), 0.0)

    return pl.pallas_call(
        body,
        grid=grid,
        in_specs=[
            pl.BlockSpec((D, 2 * D), lambda j: (0, 0)),
            pl.BlockSpec((bt, 2 * D), lambda j: (j, 0)),
        ],
        out_specs=pl.BlockSpec((D, bt), lambda j: (0, j)),
        out_shape=jax.ShapeDtypeStruct((D, b_out), jnp.float32),
    )(w, comb)


def kernel(nodes, neigh_idx, features, W_compress):
    b = nodes.shape[0]
    step = NW * G * NBUF
    b_pad = ((b + step - 1) // step) * step
    while ((b + 511) // 512) * 512 > b_pad:
        b_pad += step
    idx_all = jnp.concatenate([neigh_idx, nodes[:, None]], axis=1)
    idx_all = jnp.pad(idx_all, ((0, b_pad - b), (0, 0)))
    comb = _sc_aggregate(features, idx_all.reshape(-1), b_pad)
    return _tc_compress(comb, W_compress, b)


# speedup vs baseline: 4.2503x; 1.1332x over previous
import functools

import jax
import jax.numpy as jnp
from jax import lax
from jax.experimental import pallas as pl
from jax.experimental.pallas import tpu as pltpu
from jax.experimental.pallas import tpu_sc as plsc

D = 128
K = 10
NW = 32
G = 8
VPR = D // 16
NBUF = 4
GB = G * (K + 1) * D * 4
SB = G * D * 4


def _sc_aggregate(features, neigh_flat, nodes_pad, b_pad):
    q_per_w = b_pad // NW
    ng = q_per_w // G
    nq = ng // NBUF
    mesh = plsc.VectorSubcoreMesh(core_axis_name="c", subcore_axis_name="s")

    @functools.partial(
        pl.kernel,
        mesh=mesh,
        out_type=(
            jax.ShapeDtypeStruct((b_pad, D), jnp.float32),
            jax.ShapeDtypeStruct((b_pad, D), jnp.float32),
        ),
        scratch_types=[
            pltpu.VMEM((q_per_w * K,), jnp.int32),
            pltpu.VMEM((q_per_w,), jnp.int32),
            *[pltpu.VMEM((G * (K + 1), D), jnp.float32) for _ in range(NBUF)],
            *[pltpu.VMEM((G, D), jnp.float32) for _ in range(NBUF)],
            pltpu.SemaphoreType.DMA,
            *[pltpu.SemaphoreType.DMA for _ in range(NBUF)],
            *[pltpu.SemaphoreType.DMA for _ in range(NBUF)],
            *[pltpu.SemaphoreType.DMA for _ in range(NBUF)],
        ],
    )
    def agg(features_hbm, neigh_hbm, nodes_hbm, out_sum, out_self,
            idx_n, idx_s, *bufs):
        rows = list(bufs[0:NBUF])
        sum_v = list(bufs[NBUF:2 * NBUF])
        sem_i = bufs[2 * NBUF]
        sem_g = list(bufs[2 * NBUF + 1:2 * NBUF + 1 + NBUF])
        sem_w = list(bufs[2 * NBUF + 1 + NBUF:2 * NBUF + 1 + 2 * NBUF])
        sem_v = list(bufs[2 * NBUF + 1 + 2 * NBUF:2 * NBUF + 1 + 3 * NBUF])

        wid = lax.axis_index("s") * 2 + lax.axis_index("c")
        qbase = wid * q_per_w
        pltpu.async_copy(neigh_hbm.at[pl.ds(qbase * K, q_per_w * K)],
                         idx_n, sem_i).wait()
        pltpu.async_copy(nodes_hbm.at[pl.ds(qbase, q_per_w)],
                         idx_s, sem_i).wait()

        def gather(n, p):
            pltpu.async_copy(
                features_hbm.at[idx_n.at[pl.ds(n * (G * K), G * K)]],
                rows[p].at[pl.ds(0, G * K)], sem_g[p])
            pltpu.async_copy(
                features_hbm.at[idx_s.at[pl.ds(n * G, G)]],
                rows[p].at[pl.ds(G * K, G)], sem_g[p])

        for p in range(NBUF):
            gather(p, p)

        def ring_iter(i, carry):
            for p in range(NBUF):
                n = i * NBUF + p
                row0 = qbase + n * G
                pltpu.make_async_copy(
                    features_hbm.at[pl.ds(0, G * (K + 1))],
                    rows[p], sem_g[p]).wait()
                pltpu.async_copy(rows[p].at[pl.ds(G * K, G)],
                                 out_self.at[pl.ds(row0, G)], sem_v[p])
                @pl.when(i > 0)
                def _():
                    pltpu.make_async_copy(
                        sum_v[p], out_sum.at[pl.ds(qbase, G)],
                        sem_w[p]).wait()
                for q in range(G):
                    for v in range(VPR):
                        sl = pl.ds(v * 16, 16)
                        acc = rows[p][q * K, sl]
                        for k in range(1, K):
                            acc = acc + rows[p][q * K + k, sl]
                        sum_v[p][q, sl] = acc
                pltpu.make_async_copy(rows[p].at[pl.ds(G * K, G)],
                                      out_self.at[pl.ds(row0, G)],
                                      sem_v[p]).wait()
                @pl.when(i < nq - 1)
                def _():
                    gather(n + NBUF, p)
                pltpu.async_copy(sum_v[p], out_sum.at[pl.ds(row0, G)],
                                 sem_w[p])
            return carry

        lax.fori_loop(0, nq, ring_iter, 0)
        for p in range(NBUF):
            pltpu.make_async_copy(sum_v[p], out_sum.at[pl.ds(qbase, G)],
                                  sem_w[p]).wait()

    return agg(features, neigh_flat, nodes_pad)


def _tc_compress(sums, selfs, w, b_out):
    bt = 1024
    grid = (pl.cdiv(b_out, bt),)

    def body(w_ref, sum_ref, self_ref, o_ref):
        w_all = w_ref[...]
        wn = w_all[:, :D] * jnp.float32(1.0 / K)
        ws = w_all[:, D:]
        dn = (((1,), (1,)), ((), ()))
        o_ref[...] = jnp.maximum(
            lax.dot_general(wn, sum_ref[...], dn,
                            preferred_element_type=jnp.float32) +
            lax.dot_general(ws, self_ref[...], dn,
                            preferred_element_type=jnp.float32), 0.0)

    return pl.pallas_call(
        body,
        grid=grid,
        in_specs=[
            pl.BlockSpec((D, 2 * D), lambda j: (0, 0)),
            pl.BlockSpec((bt, D), lambda j: (j, 0)),
            pl.BlockSpec((bt, D), lambda j: (j, 0)),
        ],
        out_specs=pl.BlockSpec((D, bt), lambda j: (0, j)),
        out_shape=jax.ShapeDtypeStruct((D, b_out), jnp.float32),
    )(w, sums, selfs)


def kernel(nodes, neigh_idx, features, W_compress):
    b = nodes.shape[0]
    step = NW * G * NBUF
    b_pad = ((b + step - 1) // step) * step
    while ((b + 1023) // 1024) * 1024 > b_pad:
        b_pad += step
    neigh_flat = jnp.pad(neigh_idx.reshape(-1), (0, (b_pad - b) * K))
    nodes_pad = jnp.pad(nodes, (0, b_pad - b))
    sums, selfs = _sc_aggregate(features, neigh_flat, nodes_pad, b_pad)
    return _tc_compress(sums, selfs, W_compress, b)
